# baseline (speedup 1.0000x reference)
import functools

import jax
import jax.numpy as jnp
from jax import lax
from jax.experimental import pallas as pl
from jax.experimental.pallas import tpu as pltpu
from jax.experimental.pallas import tpu_sc as plsc

BATCH = 4096
HIST = 200
DIM = 64
TOTAL = BATCH * HIST

_INFO = plsc.get_sparse_core_info()
NC = _INFO.num_cores
NS = _INFO.num_subcores
NW = NC * NS
PER_W = TOTAL // NW
CHUNK = 512
NCHUNK = PER_W // CHUNK


def _sc_gather(idx, table):
    mesh = plsc.VectorSubcoreMesh(core_axis_name="c", subcore_axis_name="s")

    @functools.partial(
        pl.kernel,
        mesh=mesh,
        out_type=jax.ShapeDtypeStruct((TOTAL, DIM), jnp.float32),
        scratch_types=[
            pltpu.VMEM((CHUNK,), jnp.int32),
            pltpu.VMEM((CHUNK, DIM), jnp.float32),
            pltpu.SemaphoreType.DMA,
        ],
        compiler_params=pltpu.CompilerParams(use_tc_tiling_on_sc=False),
    )
    def k(idx_hbm, table_hbm, out_hbm, idx_v, rows_v, sem):
        wid = lax.axis_index("s") * NC + lax.axis_index("c")
        base = wid * PER_W

        def body(i, carry):
            off = base + i * CHUNK
            pltpu.sync_copy(idx_hbm.at[pl.ds(off, CHUNK)], idx_v)
            pltpu.async_copy(table_hbm.at[idx_v], rows_v, sem).wait()
            pltpu.sync_copy(rows_v, out_hbm.at[pl.ds(off, CHUNK)])
            return carry

        lax.fori_loop(0, NCHUNK, body, 0)

    return k(idx, table)


def kernel(channel_seq, W):
    idx = channel_seq.reshape(TOTAL).astype(jnp.int32)
    out = _sc_gather(idx, W)
    return out.reshape(BATCH, HIST, DIM)


# speedup vs baseline: 1.0473x; 1.0473x over previous
import functools

import jax
import jax.numpy as jnp
from jax import lax
from jax.experimental import pallas as pl
from jax.experimental.pallas import tpu as pltpu
from jax.experimental.pallas import tpu_sc as plsc

BATCH = 4096
HIST = 200
DIM = 64
TOTAL = BATCH * HIST

_INFO = plsc.get_sparse_core_info()
NC = _INFO.num_cores
NS = _INFO.num_subcores
NW = NC * NS
PER_W = TOTAL // NW
CHUNK = 512
NCHUNK = PER_W // CHUNK
NPAIR = NCHUNK // 2


def _sc_gather(idx, table):
    mesh = plsc.VectorSubcoreMesh(core_axis_name="c", subcore_axis_name="s")

    @functools.partial(
        pl.kernel,
        mesh=mesh,
        out_type=jax.ShapeDtypeStruct((TOTAL, DIM), jnp.float32),
        scratch_types=[
            pltpu.VMEM((PER_W,), jnp.int32),
            pltpu.VMEM((CHUNK, DIM), jnp.float32),
            pltpu.VMEM((CHUNK, DIM), jnp.float32),
            pltpu.SemaphoreType.DMA,
            pltpu.SemaphoreType.DMA,
            pltpu.SemaphoreType.DMA,
            pltpu.SemaphoreType.DMA,
        ],
        compiler_params=pltpu.CompilerParams(use_tc_tiling_on_sc=False),
    )
    def k(idx_hbm, table_hbm, out_hbm, idx_v, rows0, rows1, sg0, sg1, sw0, sw1):
        wid = lax.axis_index("s") * NC + lax.axis_index("c")
        base = wid * PER_W
        pltpu.sync_copy(idx_hbm.at[pl.ds(base, PER_W)], idx_v)

        def gather_start(g, rows, sem):
            pltpu.async_copy(table_hbm.at[idx_v.at[pl.ds(g * CHUNK, CHUNK)]],
                             rows, sem)

        def gather_wait(rows, sem):
            pltpu.make_async_copy(table_hbm.at[idx_v.at[pl.ds(0, CHUNK)]],
                                  rows, sem).wait()

        def wb_start(g, rows, sem):
            pltpu.async_copy(rows, out_hbm.at[pl.ds(base + g * CHUNK, CHUNK)],
                             sem)

        def wb_wait(rows, sem):
            pltpu.make_async_copy(rows, out_hbm.at[pl.ds(0, CHUNK)], sem).wait()

        gather_start(0, rows0, sg0)

        def pair(j, carry):
            g0 = 2 * j

            @pl.when(j > 0)
            def _():
                wb_wait(rows1, sw1)

            gather_start(g0 + 1, rows1, sg1)
            gather_wait(rows0, sg0)
            wb_start(g0, rows0, sw0)

            @pl.when(j < NPAIR - 1)
            def _():
                wb_wait(rows0, sw0)
                gather_start(g0 + 2, rows0, sg0)

            gather_wait(rows1, sg1)
            wb_start(g0 + 1, rows1, sw1)
            return carry

        lax.fori_loop(0, NPAIR, pair, 0)
        wb_wait(rows0, sw0)
        wb_wait(rows1, sw1)

    return k(idx, table)


def kernel(channel_seq, W):
    idx = channel_seq.reshape(TOTAL).astype(jnp.int32)
    out = _sc_gather(idx, W)
    return out.reshape(BATCH, HIST, DIM)
